# Initial kernel scaffold; baseline (speedup 1.0000x reference)
import jax
import jax.numpy as jnp
from jax.experimental import pallas as pl


def kernel(node_feats, edge_feats, W_in, b_in, We, be, W1, b1, W2, b2, Wp1, bp1, Wp2, bp2, edge_index, graph_ids):
    raise NotImplementedError("write your pallas kernel here")


# speedup vs baseline: 3.0307x; 3.0307x over previous
import functools

import jax
import jax.numpy as jnp
from jax import lax
from jax.experimental import pallas as pl
from jax.experimental.pallas import tpu as pltpu
from jax.experimental.pallas import tpu_sc as plsc

N = 10000
E = 320000
EMB = 128
HID = 256
L = 5
B = 64
PH = 256

NC = 2
NS = 16
NWORK = NC * NS
EPW = E // NWORK
CHUNK = 80
NCHUNK = EPW // CHUNK
ROWS_PER_TILE = 624
TAIL_ROWS = N - NS * ROWS_PER_TILE
ZROWS = 78
NLANE = 16

_vmesh = plsc.VectorSubcoreMesh(core_axis_name="c", subcore_axis_name="s")


def _sc_agg_body(h_hbm, e_hbm, src_hbm, dst_hbm, out_hbm,
                 sidx, didx, hrows, erows, zbuf, agg_sh, sem1, sem2):
    cid = lax.axis_index("c")
    sid = lax.axis_index("s")
    base = (cid * NS + sid) * EPW

    zero = jnp.zeros((NLANE,), jnp.float32)

    @pl.loop(0, ZROWS)
    def _zero_zbuf(i):
        for j in range(EMB // NLANE):
            zbuf[i, pl.ds(j * NLANE, NLANE)] = zero

    @pl.loop(0, ROWS_PER_TILE // ZROWS)
    def _zero_agg(k):
        pltpu.sync_copy(zbuf, agg_sh.at[pl.ds(sid * ROWS_PER_TILE + k * ZROWS, ZROWS)])

    @pl.when(sid == 0)
    def _zero_tail():
        pltpu.sync_copy(zbuf.at[pl.ds(0, TAIL_ROWS)],
                        agg_sh.at[pl.ds(NS * ROWS_PER_TILE, TAIL_ROWS)])

    plsc.subcore_barrier()

    @pl.loop(0, NCHUNK)
    def _chunk(k):
        off = base + k * CHUNK
        pltpu.sync_copy(src_hbm.at[pl.ds(off, CHUNK)], sidx)
        pltpu.sync_copy(dst_hbm.at[pl.ds(off, CHUNK)], didx)
        cp1 = pltpu.async_copy(h_hbm.at[sidx], hrows, sem1)
        cp2 = pltpu.async_copy(e_hbm.at[pl.ds(off, CHUNK)], erows, sem2)
        cp1.wait()
        cp2.wait()

        @pl.loop(0, CHUNK)
        def _row(i):
            for j in range(EMB // NLANE):
                sl = pl.ds(j * NLANE, NLANE)
                hrows[i, sl] = jnp.maximum(hrows[i, sl] + erows[i, sl], 0.0)

        pltpu.sync_copy(hrows, agg_sh.at[didx], add=True)

    plsc.subcore_barrier()
    pltpu.sync_copy(agg_sh.at[pl.ds(sid * ROWS_PER_TILE, ROWS_PER_TILE)],
                    out_hbm.at[cid, pl.ds(sid * ROWS_PER_TILE, ROWS_PER_TILE)])

    @pl.when(sid == 0)
    def _write_tail():
        pltpu.sync_copy(agg_sh.at[pl.ds(NS * ROWS_PER_TILE, TAIL_ROWS)],
                        out_hbm.at[cid, pl.ds(NS * ROWS_PER_TILE, TAIL_ROWS)])


_sc_agg = pl.kernel(
    _sc_agg_body,
    mesh=_vmesh,
    out_type=jax.ShapeDtypeStruct((NC, N, EMB), jnp.float32),
    scratch_types=[
        pltpu.VMEM((CHUNK,), jnp.int32),
        pltpu.VMEM((CHUNK,), jnp.int32),
        pltpu.VMEM((CHUNK, EMB), jnp.float32),
        pltpu.VMEM((CHUNK, EMB), jnp.float32),
        pltpu.VMEM((ZROWS, EMB), jnp.float32),
        pltpu.VMEM_SHARED((N, EMB), jnp.float32),
        pltpu.SemaphoreType.DMA,
        pltpu.SemaphoreType.DMA,
    ],
)


_HB = 1000
_EB = 4000
_GB = 2000
_NG = N // _GB


def _h0_body(x_ref, w_ref, b_ref, o_ref):
    o_ref[...] = (
        jnp.dot(x_ref[...], w_ref[...], preferred_element_type=jnp.float32)
        + b_ref[...]
    )


def _input_proj(x, w, b):
    return pl.pallas_call(
        _h0_body,
        grid=(N // _HB,),
        in_specs=[
            pl.BlockSpec((_HB, EMB), lambda i: (i, 0)),
            pl.BlockSpec((EMB, EMB), lambda i: (0, 0)),
            pl.BlockSpec((1, EMB), lambda i: (0, 0)),
        ],
        out_specs=pl.BlockSpec((_HB, EMB), lambda i: (i, 0)),
        out_shape=jax.ShapeDtypeStruct((N, EMB), jnp.float32),
    )(x, w, b)


def _eproj_body(ef_ref, w_ref, b_ref, o_ref):
    o_ref[...] = (
        jnp.dot(ef_ref[...], w_ref[...], preferred_element_type=jnp.float32)
        + b_ref[...]
    )


def _edge_proj(ef, w, b):
    de = ef.shape[1]
    return pl.pallas_call(
        _eproj_body,
        grid=(E // _EB,),
        in_specs=[
            pl.BlockSpec((_EB, de), lambda i: (i, 0)),
            pl.BlockSpec((de, EMB), lambda i: (0, 0)),
            pl.BlockSpec((1, EMB), lambda i: (0, 0)),
        ],
        out_specs=pl.BlockSpec((_EB, EMB), lambda i: (i, 0)),
        out_shape=jax.ShapeDtypeStruct((E, EMB), jnp.float32),
    )(ef, w, b)


def _mlp_body(final_relu, h_ref, a_ref, w1_ref, b1_ref, w2_ref, b2_ref, o_ref):
    z = h_ref[...] + a_ref[0] + a_ref[1]
    z = jnp.maximum(
        jnp.dot(z, w1_ref[...], preferred_element_type=jnp.float32) + b1_ref[...],
        0.0,
    )
    z = jnp.dot(z, w2_ref[...], preferred_element_type=jnp.float32) + b2_ref[...]
    if final_relu:
        z = jnp.maximum(z, 0.0)
    o_ref[...] = z


def _gin_mlp(h, agg, w1, b1, w2, b2, final_relu):
    return pl.pallas_call(
        functools.partial(_mlp_body, final_relu),
        grid=(N // _HB,),
        in_specs=[
            pl.BlockSpec((_HB, EMB), lambda i: (i, 0)),
            pl.BlockSpec((NC, _HB, EMB), lambda i: (0, i, 0)),
            pl.BlockSpec((EMB, HID), lambda i: (0, 0)),
            pl.BlockSpec((1, HID), lambda i: (0, 0)),
            pl.BlockSpec((HID, EMB), lambda i: (0, 0)),
            pl.BlockSpec((1, EMB), lambda i: (0, 0)),
        ],
        out_specs=pl.BlockSpec((_HB, EMB), lambda i: (i, 0)),
        out_shape=jax.ShapeDtypeStruct((N, EMB), jnp.float32),
    )(h, agg, w1, b1, w2, b2)


def _readout_body(gid_ref, h_ref, wp1_ref, bp1_ref, wp2_ref, bp2_ref, o_ref,
                  acc_ref, cnt_ref):
    i = pl.program_id(0)

    @pl.when(i == 0)
    def _():
        acc_ref[...] = jnp.zeros_like(acc_ref)
        cnt_ref[...] = jnp.zeros_like(cnt_ref)

    gid = gid_ref[0, 0, :]
    onehot = jnp.equal(
        lax.broadcasted_iota(jnp.int32, (B, _GB), 0), gid[None, :]
    ).astype(jnp.float32)
    acc_ref[...] += jnp.dot(onehot, h_ref[...], preferred_element_type=jnp.float32)
    cnt_ref[...] += jnp.broadcast_to(
        jnp.sum(onehot, axis=1, keepdims=True), (B, EMB)
    )

    @pl.when(i == _NG - 1)
    def _():
        gfeat = acc_ref[...] / jnp.maximum(cnt_ref[...], 1.0)
        p = jnp.maximum(
            jnp.dot(gfeat, wp1_ref[...], preferred_element_type=jnp.float32)
            + bp1_ref[...],
            0.0,
        )
        o_ref[...] = (
            jnp.dot(p, wp2_ref[...], preferred_element_type=jnp.float32)
            + bp2_ref[...]
        )


def _readout(gid3, h, wp1, bp1, wp2, bp2):
    return pl.pallas_call(
        _readout_body,
        grid=(_NG,),
        in_specs=[
            pl.BlockSpec((1, 1, _GB), lambda i: (i, 0, 0)),
            pl.BlockSpec((_GB, EMB), lambda i: (i, 0)),
            pl.BlockSpec((EMB, PH), lambda i: (0, 0)),
            pl.BlockSpec((1, PH), lambda i: (0, 0)),
            pl.BlockSpec((PH, 1), lambda i: (0, 0)),
            pl.BlockSpec((1, 1), lambda i: (0, 0)),
        ],
        out_specs=pl.BlockSpec((B, 1), lambda i: (0, 0)),
        out_shape=jax.ShapeDtypeStruct((B, 1), jnp.float32),
        scratch_shapes=[
            pltpu.VMEM((B, EMB), jnp.float32),
            pltpu.VMEM((B, EMB), jnp.float32),
        ],
    )(gid3, h, wp1, bp1, wp2, bp2)


def kernel(node_feats, edge_feats, W_in, b_in, We, be, W1, b1, W2, b2,
           Wp1, bp1, Wp2, bp2, edge_index, graph_ids):
    src = edge_index[0].astype(jnp.int32)
    dst = edge_index[1].astype(jnp.int32)
    gid3 = graph_ids.astype(jnp.int32).reshape(_NG, 1, _GB)

    h = _input_proj(node_feats, W_in, b_in.reshape(1, EMB))
    es = [_edge_proj(edge_feats, We[l], be[l].reshape(1, EMB)) for l in range(L)]

    for l in range(L):
        agg = _sc_agg(h, es[l], src, dst)
        h = _gin_mlp(h, agg, W1[l], b1[l].reshape(1, HID), W2[l],
                     b2[l].reshape(1, EMB), final_relu=(l < L - 1))

    return _readout(gid3, h, Wp1, bp1.reshape(1, PH), Wp2, bp2.reshape(1, 1))


# speedup vs baseline: 5.0096x; 1.6529x over previous
import functools

import jax
import jax.numpy as jnp
from jax import lax
from jax.experimental import pallas as pl
from jax.experimental.pallas import tpu as pltpu
from jax.experimental.pallas import tpu_sc as plsc

N = 10000
E = 320000
EMB = 128
HID = 256
L = 5
B = 64
PH = 256

NC = 2
NS = 16
NWORK = NC * NS
EPW = E // NWORK
CHUNK = 40
NCHUNK = EPW // CHUNK
ROWS_PER_TILE = 624
TAIL_ROWS = N - NS * ROWS_PER_TILE
NLANE = 16
NBUF = 4

_vmesh = plsc.VectorSubcoreMesh(core_axis_name="c", subcore_axis_name="s")


def _sc_agg_body(h_hbm, e_hbm, src3_hbm, dst3_hbm, out_hbm,
                 sidx, didx, hb0, eb0, hb1, eb1, hb2, eb2, hb3, eb3,
                 agg_sh,
                 is0, is1, is2, is3,
                 gs0, es0, ss0, gs1, es1, ss1, gs2, es2, ss2, gs3, es3, ss3):
    HBUF = (hb0, hb1, hb2, hb3)
    EBUF = (eb0, eb1, eb2, eb3)
    IS = (is0, is1, is2, is3)
    GS = (gs0, gs1, gs2, gs3)
    ES = (es0, es1, es2, es3)
    SS = (ss0, ss1, ss2, ss3)

    cid = lax.axis_index("c")
    sid = lax.axis_index("s")
    gwid = cid * NS + sid
    ebase = gwid * EPW

    zero = jnp.zeros((NLANE,), jnp.float32)

    @pl.loop(0, CHUNK)
    def _zero_zbuf(i):
        for j in range(EMB // NLANE):
            eb0[i, pl.ds(j * NLANE, NLANE)] = zero

    @pl.loop(0, ROWS_PER_TILE // CHUNK)
    def _zero_agg(k):
        pltpu.sync_copy(eb0, agg_sh.at[pl.ds(sid * ROWS_PER_TILE + k * CHUNK, CHUNK)])

    _rem = ROWS_PER_TILE - (ROWS_PER_TILE // CHUNK) * CHUNK
    if _rem:
        pltpu.sync_copy(
            eb0.at[pl.ds(0, _rem)],
            agg_sh.at[pl.ds(sid * ROWS_PER_TILE + ROWS_PER_TILE - _rem, _rem)])

    @pl.when(sid == 0)
    def _zero_tail():
        pltpu.sync_copy(eb0.at[pl.ds(0, TAIL_ROWS)],
                        agg_sh.at[pl.ds(NS * ROWS_PER_TILE, TAIL_ROWS)])

    plsc.subcore_barrier()

    def issue_idx(k, r):
        pltpu.async_copy(src3_hbm.at[gwid, k], sidx.at[r], IS[r])
        pltpu.async_copy(dst3_hbm.at[gwid, k], didx.at[r], IS[r])

    def wait_idx(k, r):
        pltpu.make_async_copy(src3_hbm.at[gwid, k], sidx.at[r], IS[r]).wait()
        pltpu.make_async_copy(dst3_hbm.at[gwid, k], didx.at[r], IS[r]).wait()

    def issue_io(k, b):
        pltpu.async_copy(h_hbm.at[sidx.at[b]], HBUF[b], GS[b])
        pltpu.async_copy(e_hbm.at[pl.ds(ebase + k * CHUNK, CHUNK)], EBUF[b], ES[b])

    def wait_io(k, b):
        pltpu.make_async_copy(h_hbm.at[sidx.at[b]], HBUF[b], GS[b]).wait()
        pltpu.make_async_copy(e_hbm.at[pl.ds(ebase + k * CHUNK, CHUNK)],
                              EBUF[b], ES[b]).wait()

    def compute(b):
        hb, eb = HBUF[b], EBUF[b]

        @pl.loop(0, CHUNK, step=4)
        def _row(i):
            for r in range(4):
                for j in range(EMB // NLANE):
                    sl = pl.ds(j * NLANE, NLANE)
                    hb[i + r, sl] = jnp.maximum(hb[i + r, sl] + eb[i + r, sl], 0.0)

    def issue_scatter(k, b):
        pltpu.async_copy(HBUF[b], agg_sh.at[didx.at[b]], SS[b], add=True)

    def drain_scatter(k, b):
        pltpu.make_async_copy(HBUF[b], agg_sh.at[didx.at[b]], SS[b]).wait()

    def phase(k, b, drain=True, pf_io=True, pf_idx=True):
        if drain:
            drain_scatter(k - 2, (b + 2) % NBUF)
        if pf_io:
            wait_idx(k + 1, (b + 1) % NBUF)
            issue_io(k + 1, (b + 1) % NBUF)
        if pf_idx:
            issue_idx(k + 2, (b + 2) % NBUF)
        wait_io(k, b)
        compute(b)
        issue_scatter(k, b)

    issue_idx(0, 0)
    issue_idx(1, 1)
    wait_idx(0, 0)
    issue_io(0, 0)
    phase(0, 0, drain=False)
    phase(1, 1, drain=False)

    @pl.loop(0, (NCHUNK - 6) // NBUF)
    def _steady(t):
        k = NBUF * t + 2
        phase(k, 2)
        phase(k + 1, 3)
        phase(k + 2, 0)
        phase(k + 3, 1)

    phase(NCHUNK - 4, (NCHUNK - 4) % NBUF)
    phase(NCHUNK - 3, (NCHUNK - 3) % NBUF)
    phase(NCHUNK - 2, (NCHUNK - 2) % NBUF, pf_idx=False)
    phase(NCHUNK - 1, (NCHUNK - 1) % NBUF, pf_io=False, pf_idx=False)
    drain_scatter(NCHUNK - 2, (NCHUNK - 2) % NBUF)
    drain_scatter(NCHUNK - 1, (NCHUNK - 1) % NBUF)

    plsc.subcore_barrier()
    pltpu.sync_copy(agg_sh.at[pl.ds(sid * ROWS_PER_TILE, ROWS_PER_TILE)],
                    out_hbm.at[cid, pl.ds(sid * ROWS_PER_TILE, ROWS_PER_TILE)])

    @pl.when(sid == 0)
    def _write_tail():
        pltpu.sync_copy(agg_sh.at[pl.ds(NS * ROWS_PER_TILE, TAIL_ROWS)],
                        out_hbm.at[cid, pl.ds(NS * ROWS_PER_TILE, TAIL_ROWS)])


_sc_agg = pl.kernel(
    _sc_agg_body,
    mesh=_vmesh,
    out_type=jax.ShapeDtypeStruct((NC, N, EMB), jnp.float32),
    scratch_types=[
        pltpu.VMEM((NBUF, CHUNK), jnp.int32),
        pltpu.VMEM((NBUF, CHUNK), jnp.int32),
    ] + [pltpu.VMEM((CHUNK, EMB), jnp.float32)] * (2 * NBUF) + [
        pltpu.VMEM_SHARED((N, EMB), jnp.float32),
    ] + [pltpu.SemaphoreType.DMA] * 16,
)


_HB = 1000
_EB = 4000
_GB = 2000
_NG = N // _GB


def _h0_body(x_ref, w_ref, b_ref, o_ref):
    o_ref[...] = (
        jnp.dot(x_ref[...], w_ref[...], preferred_element_type=jnp.float32)
        + b_ref[...]
    )


def _input_proj(x, w, b):
    return pl.pallas_call(
        _h0_body,
        grid=(N // _HB,),
        in_specs=[
            pl.BlockSpec((_HB, EMB), lambda i: (i, 0)),
            pl.BlockSpec((EMB, EMB), lambda i: (0, 0)),
            pl.BlockSpec((1, EMB), lambda i: (0, 0)),
        ],
        out_specs=pl.BlockSpec((_HB, EMB), lambda i: (i, 0)),
        out_shape=jax.ShapeDtypeStruct((N, EMB), jnp.float32),
    )(x, w, b)


def _eproj_body(ef_ref, w_ref, b_ref, o_ref):
    o_ref[...] = (
        jnp.dot(ef_ref[...], w_ref[...], preferred_element_type=jnp.float32)
        + b_ref[...]
    )


def _edge_proj(ef, w, b):
    de = ef.shape[1]
    return pl.pallas_call(
        _eproj_body,
        grid=(E // _EB,),
        in_specs=[
            pl.BlockSpec((_EB, de), lambda i: (i, 0)),
            pl.BlockSpec((de, EMB), lambda i: (0, 0)),
            pl.BlockSpec((1, EMB), lambda i: (0, 0)),
        ],
        out_specs=pl.BlockSpec((_EB, EMB), lambda i: (i, 0)),
        out_shape=jax.ShapeDtypeStruct((E, EMB), jnp.float32),
    )(ef, w, b)


def _mlp_body(final_relu, h_ref, a_ref, w1_ref, b1_ref, w2_ref, b2_ref, o_ref):
    z = h_ref[...] + a_ref[0] + a_ref[1]
    z = jnp.maximum(
        jnp.dot(z, w1_ref[...], preferred_element_type=jnp.float32) + b1_ref[...],
        0.0,
    )
    z = jnp.dot(z, w2_ref[...], preferred_element_type=jnp.float32) + b2_ref[...]
    if final_relu:
        z = jnp.maximum(z, 0.0)
    o_ref[...] = z


def _gin_mlp(h, agg, w1, b1, w2, b2, final_relu):
    return pl.pallas_call(
        functools.partial(_mlp_body, final_relu),
        grid=(N // _HB,),
        in_specs=[
            pl.BlockSpec((_HB, EMB), lambda i: (i, 0)),
            pl.BlockSpec((NC, _HB, EMB), lambda i: (0, i, 0)),
            pl.BlockSpec((EMB, HID), lambda i: (0, 0)),
            pl.BlockSpec((1, HID), lambda i: (0, 0)),
            pl.BlockSpec((HID, EMB), lambda i: (0, 0)),
            pl.BlockSpec((1, EMB), lambda i: (0, 0)),
        ],
        out_specs=pl.BlockSpec((_HB, EMB), lambda i: (i, 0)),
        out_shape=jax.ShapeDtypeStruct((N, EMB), jnp.float32),
    )(h, agg, w1, b1, w2, b2)


def _readout_body(gid_ref, h_ref, wp1_ref, bp1_ref, wp2_ref, bp2_ref, o_ref,
                  acc_ref, cnt_ref):
    i = pl.program_id(0)

    @pl.when(i == 0)
    def _():
        acc_ref[...] = jnp.zeros_like(acc_ref)
        cnt_ref[...] = jnp.zeros_like(cnt_ref)

    gid = gid_ref[0, 0, :]
    onehot = jnp.equal(
        lax.broadcasted_iota(jnp.int32, (B, _GB), 0), gid[None, :]
    ).astype(jnp.float32)
    acc_ref[...] += jnp.dot(onehot, h_ref[...], preferred_element_type=jnp.float32)
    cnt_ref[...] += jnp.broadcast_to(
        jnp.sum(onehot, axis=1, keepdims=True), (B, EMB)
    )

    @pl.when(i == _NG - 1)
    def _():
        gfeat = acc_ref[...] / jnp.maximum(cnt_ref[...], 1.0)
        p = jnp.maximum(
            jnp.dot(gfeat, wp1_ref[...], preferred_element_type=jnp.float32)
            + bp1_ref[...],
            0.0,
        )
        o_ref[...] = (
            jnp.dot(p, wp2_ref[...], preferred_element_type=jnp.float32)
            + bp2_ref[...]
        )


def _readout(gid3, h, wp1, bp1, wp2, bp2):
    return pl.pallas_call(
        _readout_body,
        grid=(_NG,),
        in_specs=[
            pl.BlockSpec((1, 1, _GB), lambda i: (i, 0, 0)),
            pl.BlockSpec((_GB, EMB), lambda i: (i, 0)),
            pl.BlockSpec((EMB, PH), lambda i: (0, 0)),
            pl.BlockSpec((1, PH), lambda i: (0, 0)),
            pl.BlockSpec((PH, 1), lambda i: (0, 0)),
            pl.BlockSpec((1, 1), lambda i: (0, 0)),
        ],
        out_specs=pl.BlockSpec((B, 1), lambda i: (0, 0)),
        out_shape=jax.ShapeDtypeStruct((B, 1), jnp.float32),
        scratch_shapes=[
            pltpu.VMEM((B, EMB), jnp.float32),
            pltpu.VMEM((B, EMB), jnp.float32),
        ],
    )(gid3, h, wp1, bp1, wp2, bp2)


def kernel(node_feats, edge_feats, W_in, b_in, We, be, W1, b1, W2, b2,
           Wp1, bp1, Wp2, bp2, edge_index, graph_ids):
    src3 = edge_index[0].astype(jnp.int32).reshape(NWORK, NCHUNK, CHUNK)
    dst3 = edge_index[1].astype(jnp.int32).reshape(NWORK, NCHUNK, CHUNK)
    gid3 = graph_ids.astype(jnp.int32).reshape(_NG, 1, _GB)

    h = _input_proj(node_feats, W_in, b_in.reshape(1, EMB))
    es = [_edge_proj(edge_feats, We[l], be[l].reshape(1, EMB)) for l in range(L)]

    for l in range(L):
        agg = _sc_agg(h, es[l], src3, dst3)
        h = _gin_mlp(h, agg, W1[l], b1[l].reshape(1, HID), W2[l],
                     b2[l].reshape(1, EMB), final_relu=(l < L - 1))

    return _readout(gid3, h, Wp1, bp1.reshape(1, PH), Wp2, bp2.reshape(1, 1))


# speedup vs baseline: 6.1111x; 1.2199x over previous
import functools

import jax
import jax.numpy as jnp
from jax import lax
from jax.experimental import pallas as pl
from jax.experimental.pallas import tpu as pltpu
from jax.experimental.pallas import tpu_sc as plsc

N = 10000
E = 320000
EMB = 128
HID = 256
L = 5
B = 64
PH = 256

NC = 2
NS = 16
NWORK = NC * NS
EPW = E // NWORK
CHUNK = 40
NCHUNK = EPW // CHUNK
ROWS_PER_TILE = 624
TAIL_ROWS = N - NS * ROWS_PER_TILE
NLANE = 16
NBUF = 4

_vmesh = plsc.VectorSubcoreMesh(core_axis_name="c", subcore_axis_name="s")


def _sc_agg_body(h_hbm, e_hbm, src3_hbm, dst3_hbm, out_hbm,
                 sidx, didx, hb0, eb0, hb1, eb1, hb2, eb2, hb3, eb3,
                 agg_sh,
                 is0, is1, is2, is3,
                 gs0, es0, ss0, gs1, es1, ss1, gs2, es2, ss2, gs3, es3, ss3):
    HBUF = (hb0, hb1, hb2, hb3)
    EBUF = (eb0, eb1, eb2, eb3)
    IS = (is0, is1, is2, is3)
    GS = (gs0, gs1, gs2, gs3)
    ES = (es0, es1, es2, es3)
    SS = (ss0, ss1, ss2, ss3)

    cid = lax.axis_index("c")
    sid = lax.axis_index("s")
    gwid = cid * NS + sid
    ebase = gwid * EPW

    zero = jnp.zeros((NLANE,), jnp.float32)

    @pl.loop(0, CHUNK)
    def _zero_zbuf(i):
        for j in range(EMB // NLANE):
            eb0[i, pl.ds(j * NLANE, NLANE)] = zero

    @pl.loop(0, ROWS_PER_TILE // CHUNK)
    def _zero_agg(k):
        pltpu.sync_copy(eb0, agg_sh.at[pl.ds(sid * ROWS_PER_TILE + k * CHUNK, CHUNK)])

    _rem = ROWS_PER_TILE - (ROWS_PER_TILE // CHUNK) * CHUNK
    if _rem:
        pltpu.sync_copy(
            eb0.at[pl.ds(0, _rem)],
            agg_sh.at[pl.ds(sid * ROWS_PER_TILE + ROWS_PER_TILE - _rem, _rem)])

    @pl.when(sid == 0)
    def _zero_tail():
        pltpu.sync_copy(eb0.at[pl.ds(0, TAIL_ROWS)],
                        agg_sh.at[pl.ds(NS * ROWS_PER_TILE, TAIL_ROWS)])

    plsc.subcore_barrier()

    def issue_idx(k, r):
        off = ebase + k * CHUNK
        pltpu.async_copy(src3_hbm.at[pl.ds(off, CHUNK)], sidx.at[r], IS[r])
        pltpu.async_copy(dst3_hbm.at[pl.ds(off, CHUNK)], didx.at[r], IS[r])

    def wait_idx(k, r):
        off = ebase + k * CHUNK
        pltpu.make_async_copy(src3_hbm.at[pl.ds(off, CHUNK)], sidx.at[r], IS[r]).wait()
        pltpu.make_async_copy(dst3_hbm.at[pl.ds(off, CHUNK)], didx.at[r], IS[r]).wait()

    def issue_io(k, b):
        pltpu.async_copy(h_hbm.at[sidx.at[b]], HBUF[b], GS[b])
        pltpu.async_copy(e_hbm.at[pl.ds(ebase + k * CHUNK, CHUNK)], EBUF[b], ES[b])

    def wait_io(k, b):
        pltpu.make_async_copy(h_hbm.at[sidx.at[b]], HBUF[b], GS[b]).wait()
        pltpu.make_async_copy(e_hbm.at[pl.ds(ebase + k * CHUNK, CHUNK)],
                              EBUF[b], ES[b]).wait()

    def compute(b):
        hb, eb = HBUF[b], EBUF[b]

        @pl.loop(0, CHUNK, step=4)
        def _row(i):
            for r in range(4):
                for j in range(EMB // NLANE):
                    sl = pl.ds(j * NLANE, NLANE)
                    hb[i + r, sl] = jnp.maximum(hb[i + r, sl] + eb[i + r, sl], 0.0)

    def issue_scatter(k, b):
        pltpu.async_copy(HBUF[b], agg_sh.at[didx.at[b]], SS[b], add=True)

    def drain_scatter(k, b):
        pltpu.make_async_copy(HBUF[b], agg_sh.at[didx.at[b]], SS[b]).wait()

    def phase(k, b, drain=True, pf_io=True, pf_idx=True):
        if drain:
            drain_scatter(k - 2, (b + 2) % NBUF)
        if pf_io:
            wait_idx(k + 1, (b + 1) % NBUF)
            issue_io(k + 1, (b + 1) % NBUF)
        if pf_idx:
            issue_idx(k + 2, (b + 2) % NBUF)
        wait_io(k, b)
        compute(b)
        issue_scatter(k, b)

    issue_idx(0, 0)
    issue_idx(1, 1)
    wait_idx(0, 0)
    issue_io(0, 0)
    phase(0, 0, drain=False)
    phase(1, 1, drain=False)

    @pl.loop(0, (NCHUNK - 6) // NBUF)
    def _steady(t):
        k = NBUF * t + 2
        phase(k, 2)
        phase(k + 1, 3)
        phase(k + 2, 0)
        phase(k + 3, 1)

    phase(NCHUNK - 4, (NCHUNK - 4) % NBUF)
    phase(NCHUNK - 3, (NCHUNK - 3) % NBUF)
    phase(NCHUNK - 2, (NCHUNK - 2) % NBUF, pf_idx=False)
    phase(NCHUNK - 1, (NCHUNK - 1) % NBUF, pf_io=False, pf_idx=False)
    drain_scatter(NCHUNK - 2, (NCHUNK - 2) % NBUF)
    drain_scatter(NCHUNK - 1, (NCHUNK - 1) % NBUF)

    plsc.subcore_barrier()
    pltpu.sync_copy(agg_sh.at[pl.ds(sid * ROWS_PER_TILE, ROWS_PER_TILE)],
                    out_hbm.at[cid, pl.ds(sid * ROWS_PER_TILE, ROWS_PER_TILE)])

    @pl.when(sid == 0)
    def _write_tail():
        pltpu.sync_copy(agg_sh.at[pl.ds(NS * ROWS_PER_TILE, TAIL_ROWS)],
                        out_hbm.at[cid, pl.ds(NS * ROWS_PER_TILE, TAIL_ROWS)])


_sc_agg = pl.kernel(
    _sc_agg_body,
    mesh=_vmesh,
    out_type=jax.ShapeDtypeStruct((NC, N, EMB), jnp.float32),
    scratch_types=[
        pltpu.VMEM((NBUF, CHUNK), jnp.int32),
        pltpu.VMEM((NBUF, CHUNK), jnp.int32),
    ] + [pltpu.VMEM((CHUNK, EMB), jnp.float32)] * (2 * NBUF) + [
        pltpu.VMEM_SHARED((N, EMB), jnp.float32),
    ] + [pltpu.SemaphoreType.DMA] * 16,
)


_HB = 1000
_EB = 6400
_GB = 2000
_NG = N // _GB


def _h0_body(x_ref, w_ref, b_ref, o_ref):
    o_ref[...] = (
        jnp.dot(x_ref[...], w_ref[...], preferred_element_type=jnp.float32)
        + b_ref[...]
    )


def _input_proj(x, w, b):
    return pl.pallas_call(
        _h0_body,
        grid=(N // _HB,),
        in_specs=[
            pl.BlockSpec((_HB, EMB), lambda i: (i, 0)),
            pl.BlockSpec((EMB, EMB), lambda i: (0, 0)),
            pl.BlockSpec((1, EMB), lambda i: (0, 0)),
        ],
        out_specs=pl.BlockSpec((_HB, EMB), lambda i: (i, 0)),
        out_shape=jax.ShapeDtypeStruct((N, EMB), jnp.float32),
    )(x, w, b)


def _eproj_body(eft_ref, w_ref, b_ref, o_ref):
    o_ref[...] = (
        lax.dot_general(eft_ref[...], w_ref[...], (((0,), (0,)), ((), ())),
                        preferred_element_type=jnp.float32)
        + b_ref[...]
    )


def _edge_proj(eft, w, b):
    de = eft.shape[0]
    return pl.pallas_call(
        _eproj_body,
        grid=(E // _EB,),
        in_specs=[
            pl.BlockSpec((de, _EB), lambda i: (0, i)),
            pl.BlockSpec((de, EMB), lambda i: (0, 0)),
            pl.BlockSpec((1, EMB), lambda i: (0, 0)),
        ],
        out_specs=pl.BlockSpec((_EB, EMB), lambda i: (i, 0)),
        out_shape=jax.ShapeDtypeStruct((E, EMB), jnp.float32),
    )(eft, w, b)


def _mlp_body(final_relu, h_ref, a_ref, w1_ref, b1_ref, w2_ref, b2_ref, o_ref):
    z = h_ref[...] + (a_ref[0] + a_ref[1])
    z = jnp.maximum(
        jnp.dot(z, w1_ref[...], preferred_element_type=jnp.float32) + b1_ref[...],
        0.0,
    )
    z = jnp.dot(z, w2_ref[...], preferred_element_type=jnp.float32) + b2_ref[...]
    if final_relu:
        z = jnp.maximum(z, 0.0)
    o_ref[...] = z


def _gin_mlp(h, agg, w1, b1, w2, b2, final_relu):
    return pl.pallas_call(
        functools.partial(_mlp_body, final_relu),
        grid=(N // _HB,),
        in_specs=[
            pl.BlockSpec((_HB, EMB), lambda i: (i, 0)),
            pl.BlockSpec((NC, _HB, EMB), lambda i: (0, i, 0)),
            pl.BlockSpec((EMB, HID), lambda i: (0, 0)),
            pl.BlockSpec((1, HID), lambda i: (0, 0)),
            pl.BlockSpec((HID, EMB), lambda i: (0, 0)),
            pl.BlockSpec((1, EMB), lambda i: (0, 0)),
        ],
        out_specs=pl.BlockSpec((_HB, EMB), lambda i: (i, 0)),
        out_shape=jax.ShapeDtypeStruct((N, EMB), jnp.float32),
    )(h, agg, w1, b1, w2, b2)


def _readout_body(gid_ref, h_ref, wp1_ref, bp1_ref, wp2_ref, bp2_ref, o_ref,
                  acc_ref, cnt_ref):
    i = pl.program_id(0)

    @pl.when(i == 0)
    def _():
        acc_ref[...] = jnp.zeros_like(acc_ref)
        cnt_ref[...] = jnp.zeros_like(cnt_ref)

    gid = gid_ref[0, 0, :]
    onehot = jnp.equal(
        lax.broadcasted_iota(jnp.int32, (B, _GB), 0), gid[None, :]
    ).astype(jnp.float32)
    acc_ref[...] += jnp.dot(onehot, h_ref[...], preferred_element_type=jnp.float32)
    cnt_ref[...] += jnp.broadcast_to(
        jnp.sum(onehot, axis=1, keepdims=True), (B, EMB)
    )

    @pl.when(i == _NG - 1)
    def _():
        gfeat = acc_ref[...] / jnp.maximum(cnt_ref[...], 1.0)
        p = jnp.maximum(
            jnp.dot(gfeat, wp1_ref[...], preferred_element_type=jnp.float32)
            + bp1_ref[...],
            0.0,
        )
        o_ref[...] = (
            jnp.dot(p, wp2_ref[...], preferred_element_type=jnp.float32)
            + bp2_ref[...]
        )


def _readout(gid3, h, wp1, bp1, wp2, bp2):
    return pl.pallas_call(
        _readout_body,
        grid=(_NG,),
        in_specs=[
            pl.BlockSpec((1, 1, _GB), lambda i: (i, 0, 0)),
            pl.BlockSpec((_GB, EMB), lambda i: (i, 0)),
            pl.BlockSpec((EMB, PH), lambda i: (0, 0)),
            pl.BlockSpec((1, PH), lambda i: (0, 0)),
            pl.BlockSpec((PH, 1), lambda i: (0, 0)),
            pl.BlockSpec((1, 1), lambda i: (0, 0)),
        ],
        out_specs=pl.BlockSpec((B, 1), lambda i: (0, 0)),
        out_shape=jax.ShapeDtypeStruct((B, 1), jnp.float32),
        scratch_shapes=[
            pltpu.VMEM((B, EMB), jnp.float32),
            pltpu.VMEM((B, EMB), jnp.float32),
        ],
    )(gid3, h, wp1, bp1, wp2, bp2)


def kernel(node_feats, edge_feats, W_in, b_in, We, be, W1, b1, W2, b2,
           Wp1, bp1, Wp2, bp2, edge_index, graph_ids):
    src = edge_index[0].astype(jnp.int32)
    dst = edge_index[1].astype(jnp.int32)
    gid3 = graph_ids.astype(jnp.int32).reshape(_NG, 1, _GB)
    eft = edge_feats.T

    h = _input_proj(node_feats, W_in, b_in.reshape(1, EMB))
    es = [_edge_proj(eft, We[l], be[l].reshape(1, EMB)) for l in range(L)]

    for l in range(L):
        agg = _sc_agg(h, es[l], src, dst)
        h = _gin_mlp(h, agg, W1[l], b1[l].reshape(1, HID), W2[l],
                     b2[l].reshape(1, EMB), final_relu=(l < L - 1))

    return _readout(gid3, h, Wp1, bp1.reshape(1, PH), Wp2, bp2.reshape(1, 1))
